# baseline (speedup 1.0000x reference)
import functools

import jax
import jax.numpy as jnp
from jax import lax
from jax.experimental import pallas as pl
from jax.experimental.pallas import tpu as pltpu
from jax.experimental.pallas import tpu_sc as plsc

N = 10000
E = 320000
D = 128
H = 128
G = 128
OUT = 10
EPS = 1e-5

NC = 2
NS = 16
NW = NC * NS
CH = 128
PT = 10240
NCHUNK = PT // CH
HF = CH // 2
EPAD = PT * NW
NPAD = 10112
ZR = NPAD // NS
DW = 128

@functools.cache
def _sc_kernels():
    mesh = plsc.VectorSubcoreMesh(core_axis_name="c", subcore_axis_name="s",
                                  num_cores=NC, num_subcores=NS)
    deg = functools.partial(
        pl.kernel,
        out_type=jax.ShapeDtypeStruct((NC, NPAD, DW), jnp.float32),
        mesh=mesh,
        scratch_types=[
            pltpu.VMEM((NCHUNK, CH), jnp.int32),
            pltpu.VMEM((CH, DW), jnp.float32),
            pltpu.VMEM_SHARED((NPAD, DW), jnp.float32),
        ],
    )(_sc_degree_body)
    agg = functools.partial(
        pl.kernel,
        out_type=jax.ShapeDtypeStruct((NC, NPAD, H), jnp.float32),
        mesh=mesh,
        scratch_types=[
            pltpu.VMEM((NCHUNK, CH), jnp.int32),
            pltpu.VMEM((NCHUNK, CH), jnp.int32),
            pltpu.VMEM((CH, H), jnp.float32),
            pltpu.VMEM_SHARED((NPAD, H), jnp.float32),
            pltpu.SemaphoreType.DMA,
        ],
    )(_sc_aggregate_body)
    return deg, agg


def _sc_degree_body(dst_hbm, ones_hbm, zeros_hbm, out_hbm, dst_v, ones_v, acc):
    c = lax.axis_index("c")
    s = lax.axis_index("s")
    wid = s * NC + c
    pltpu.sync_copy(dst_hbm.at[wid], dst_v)
    pltpu.sync_copy(ones_hbm, ones_v)
    pltpu.sync_copy(zeros_hbm.at[pl.ds(s * ZR, ZR)], acc.at[pl.ds(s * ZR, ZR)])
    plsc.subcore_barrier()

    def body(j, carry):
        pltpu.sync_copy(ones_v, acc.at[dst_v.at[j]], add=True)
        return carry

    lax.fori_loop(0, NCHUNK, body, 0)
    plsc.subcore_barrier()
    pltpu.sync_copy(acc.at[pl.ds(s * ZR, ZR)], out_hbm.at[c].at[pl.ds(s * ZR, ZR)])


def _sc_aggregate_body(y_hbm, src_hbm, dst_hbm, zeros_hbm, out_hbm,
                       src_v, dst_v, buf_a, acc, sem_a):
    c = lax.axis_index("c")
    s = lax.axis_index("s")
    wid = s * NC + c
    pltpu.sync_copy(src_hbm.at[wid], src_v)
    pltpu.sync_copy(dst_hbm.at[wid], dst_v)
    pltpu.sync_copy(zeros_hbm.at[pl.ds(s * ZR, ZR)], acc.at[pl.ds(s * ZR, ZR)])
    plsc.subcore_barrier()

    def body(k, carry):
        pltpu.async_copy(y_hbm.at[src_v.at[k]], buf_a, sem_a).wait()
        pltpu.sync_copy(buf_a, acc.at[dst_v.at[k]], add=True)
        return carry

    lax.fori_loop(0, NCHUNK, body, 0)
    plsc.subcore_barrier()
    pltpu.sync_copy(acc.at[pl.ds(s * ZR, ZR)], out_hbm.at[c].at[pl.ds(s * ZR, ZR)])


def _tc_front_body(x_ref, win_ref, bin_ref, wc0_ref, degp_ref,
                   h0_ref, y0_ref, dinv_ref):
    degp = degp_ref[...]
    deg = degp[0, :N, 0] + degp[1, :N, 0] + 1.0
    dinv = lax.rsqrt(jnp.maximum(deg, 1.0))
    h0 = jnp.maximum(x_ref[...] @ win_ref[...] + bin_ref[...], 0.0)
    h0_ref[...] = h0
    y0_ref[...] = (h0 @ wc0_ref[...]) * dinv[:, None]
    dinv_ref[...] = dinv[:, None]


def _tc_layer_body(has_next, parts_ref, y_ref, h_ref, dinv_ref, b_ref,
                   g_ref, be_ref, batch_ref, wnext_ref,
                   hn_ref, pool_ref, ynext_ref):
    parts = parts_ref[...]
    dinv = dinv_ref[...]
    agg = (parts[0, :N] + parts[1, :N] + y_ref[...]) * dinv + b_ref[...]
    mu = jnp.mean(agg, axis=0, keepdims=True)
    var = jnp.mean((agg - mu) ** 2, axis=0, keepdims=True)
    hnew = (agg - mu) * lax.rsqrt(var + EPS) * g_ref[...] + be_ref[...]
    hnew = h_ref[...] + jnp.maximum(hnew, 0.0)
    hn_ref[...] = hnew
    seg = jnp.arange(G, dtype=jnp.int32)[:, None] == batch_ref[...]
    pool_ref[...] = jnp.dot(seg.astype(jnp.float32), hnew,
                            preferred_element_type=jnp.float32)
    if has_next:
        ynext_ref[...] = (hnew @ wnext_ref[...]) * dinv


def _tc_head_body(p0_ref, p1_ref, p2_ref, wjk_ref, bjk_ref, wh1_ref,
                  bh1_ref, wh2_ref, bh2_ref, out_ref):
    z = jnp.concatenate([p0_ref[...], p1_ref[...], p2_ref[...]], axis=1)
    z = jnp.maximum(z @ wjk_ref[...] + bjk_ref[...], 0.0)
    z = jnp.maximum(z @ wh1_ref[...] + bh1_ref[...], 0.0)
    out_ref[...] = z @ wh2_ref[...] + bh2_ref[...]


def _tc_call(body, out_shapes, *args):
    return pl.pallas_call(
        body,
        out_shape=out_shapes,
    )(*args)


def kernel(x, edge_index, batch, W_in, b_in, Wc0, bc0, g0, be0, Wc1, bc1,
           g1, be1, Wc2, bc2, g2, be2, W_jk, b_jk, Wh1, bh1, Wh2, bh2):
    f32 = jnp.float32
    pad = EPAD - E
    src_p = jnp.concatenate(
        [edge_index[0], jnp.zeros((pad,), jnp.int32)]).reshape(NW, NCHUNK, CH)
    pad_dst = N + (jnp.arange(pad, dtype=jnp.int32) % (NPAD - N))
    dst_p = jnp.concatenate(
        [edge_index[1], pad_dst]).reshape(NW, NCHUNK, CH)
    zeros_h = jnp.zeros((NPAD, H), f32)
    ones_d = jnp.ones((CH, DW), f32)
    batch_row = batch.reshape(1, N)

    _sc_degree, _sc_aggregate = _sc_kernels()
    deg_parts = _sc_degree(dst_p, ones_d, zeros_h)
    h0, y0, dinv = _tc_call(
        _tc_front_body,
        (jax.ShapeDtypeStruct((N, H), f32),
         jax.ShapeDtypeStruct((N, H), f32),
         jax.ShapeDtypeStruct((N, 1), f32)),
        x, W_in, b_in.reshape(1, H), Wc0, deg_parts)

    convs = ((bc0, g0, be0, Wc1), (bc1, g1, be1, Wc2), (bc2, g2, be2, Wc2))
    h, y = h0, y0
    pools = []
    for li, (b, g, be, wnext) in enumerate(convs):
        parts = _sc_aggregate(y, src_p, dst_p, zeros_h)
        has_next = li < 2
        outs = _tc_call(
            functools.partial(_tc_layer_body, has_next),
            (jax.ShapeDtypeStruct((N, H), f32),
             jax.ShapeDtypeStruct((G, H), f32),
             jax.ShapeDtypeStruct((N, H), f32)),
            parts, y, h, dinv, b.reshape(1, H), g.reshape(1, H),
            be.reshape(1, H), batch_row, wnext)
        h, pool, y = outs
        pools.append(pool)

    out = _tc_call(
        _tc_head_body,
        jax.ShapeDtypeStruct((G, OUT), f32),
        pools[0], pools[1], pools[2], W_jk, b_jk.reshape(1, H),
        Wh1, bh1.reshape(1, H), Wh2, bh2.reshape(1, OUT))
    return out


# speedup vs baseline: 1.4779x; 1.4779x over previous
import functools

import jax
import jax.numpy as jnp
from jax import lax
from jax.experimental import pallas as pl
from jax.experimental.pallas import tpu as pltpu
from jax.experimental.pallas import tpu_sc as plsc

N = 10000
E = 320000
D = 128
H = 128
G = 128
OUT = 10
EPS = 1e-5

NC = 2
NS = 16
NW = NC * NS
CH = 128
PT = 10112
NCHUNK = PT // CH
EPAD = PT * NW
NPAD = 10112
ZR = NPAD // NS
DW = 128

@functools.cache
def _sc_kernels():
    mesh = plsc.VectorSubcoreMesh(core_axis_name="c", subcore_axis_name="s",
                                  num_cores=NC, num_subcores=NS)
    deg = functools.partial(
        pl.kernel,
        out_type=jax.ShapeDtypeStruct((NC, NPAD, DW), jnp.float32),
        mesh=mesh,
        scratch_types=[
            pltpu.VMEM((NCHUNK, CH), jnp.int32),
            pltpu.VMEM((CH, DW), jnp.float32),
            pltpu.VMEM_SHARED((NPAD, DW), jnp.float32),
        ],
    )(_sc_degree_body)
    agg = functools.partial(
        pl.kernel,
        out_type=jax.ShapeDtypeStruct((NC, NPAD, H), jnp.float32),
        mesh=mesh,
        scratch_types=[
            pltpu.VMEM((NCHUNK, CH), jnp.int32),
            pltpu.VMEM((NCHUNK, CH), jnp.int32),
            pltpu.VMEM((CH, H), jnp.float32),
            pltpu.VMEM_SHARED((NPAD, H), jnp.float32),
            pltpu.SemaphoreType.DMA,
        ],
    )(_sc_aggregate_body)
    return deg, agg


def _sc_degree_body(dst_hbm, ones_hbm, zeros_hbm, out_hbm, dst_v, ones_v, acc):
    c = lax.axis_index("c")
    s = lax.axis_index("s")
    wid = s * NC + c
    pltpu.sync_copy(dst_hbm.at[wid], dst_v)
    pltpu.sync_copy(ones_hbm, ones_v)
    pltpu.sync_copy(zeros_hbm.at[pl.ds(s * ZR, ZR)], acc.at[pl.ds(s * ZR, ZR)])
    plsc.subcore_barrier()

    def body(j, carry):
        pltpu.sync_copy(ones_v, acc.at[dst_v.at[j]], add=True)
        return carry

    lax.fori_loop(0, NCHUNK, body, 0)
    plsc.subcore_barrier()
    pltpu.sync_copy(acc.at[pl.ds(s * ZR, ZR)], out_hbm.at[c].at[pl.ds(s * ZR, ZR)])


def _sc_aggregate_body(y_hbm, src_hbm, dst_hbm, zeros_hbm, out_hbm,
                       src_v, dst_v, buf_a, acc, sem_a):
    c = lax.axis_index("c")
    s = lax.axis_index("s")
    wid = s * NC + c
    pltpu.sync_copy(src_hbm.at[wid], src_v)
    pltpu.sync_copy(dst_hbm.at[wid], dst_v)
    pltpu.sync_copy(zeros_hbm.at[pl.ds(s * ZR, ZR)], acc.at[pl.ds(s * ZR, ZR)])
    plsc.subcore_barrier()

    def body(k, carry):
        pltpu.async_copy(y_hbm.at[src_v.at[k]], buf_a, sem_a).wait()
        pltpu.sync_copy(buf_a, acc.at[dst_v.at[k]], add=True)
        return carry

    lax.fori_loop(0, NCHUNK, body, 0)
    plsc.subcore_barrier()
    pltpu.sync_copy(acc.at[pl.ds(s * ZR, ZR)], out_hbm.at[c].at[pl.ds(s * ZR, ZR)])


def _tc_front_body(x_ref, win_ref, bin_ref, wc0_ref, degp_ref,
                   h0_ref, y0_ref, dinv_ref):
    degp = degp_ref[...]
    deg = degp[0, :N, 0] + degp[1, :N, 0] + 1.0
    dinv = lax.rsqrt(jnp.maximum(deg, 1.0))
    h0 = jnp.maximum(x_ref[...] @ win_ref[...] + bin_ref[...], 0.0)
    h0_ref[...] = h0
    y0_ref[...] = (h0 @ wc0_ref[...]) * dinv[:, None]
    dinv_ref[...] = dinv[:, None]


def _tc_layer_body(has_next, parts_ref, y_ref, h_ref, dinv_ref, b_ref,
                   g_ref, be_ref, batch_ref, wnext_ref,
                   hn_ref, pool_ref, ynext_ref):
    parts = parts_ref[...]
    dinv = dinv_ref[...]
    agg = (parts[0, :N] + parts[1, :N] + y_ref[...]) * dinv + b_ref[...]
    mu = jnp.mean(agg, axis=0, keepdims=True)
    var = jnp.mean((agg - mu) ** 2, axis=0, keepdims=True)
    hnew = (agg - mu) * lax.rsqrt(var + EPS) * g_ref[...] + be_ref[...]
    hnew = h_ref[...] + jnp.maximum(hnew, 0.0)
    hn_ref[...] = hnew
    seg = jnp.arange(G, dtype=jnp.int32)[:, None] == batch_ref[...]
    pool_ref[...] = jnp.dot(seg.astype(jnp.float32), hnew,
                            preferred_element_type=jnp.float32)
    if has_next:
        ynext_ref[...] = (hnew @ wnext_ref[...]) * dinv


def _tc_head_body(p0_ref, p1_ref, p2_ref, wjk_ref, bjk_ref, wh1_ref,
                  bh1_ref, wh2_ref, bh2_ref, out_ref):
    z = jnp.concatenate([p0_ref[...], p1_ref[...], p2_ref[...]], axis=1)
    z = jnp.maximum(z @ wjk_ref[...] + bjk_ref[...], 0.0)
    z = jnp.maximum(z @ wh1_ref[...] + bh1_ref[...], 0.0)
    out_ref[...] = z @ wh2_ref[...] + bh2_ref[...]


def _tc_call(body, out_shapes, *args):
    return pl.pallas_call(
        body,
        out_shape=out_shapes,
    )(*args)


def kernel(x, edge_index, batch, W_in, b_in, Wc0, bc0, g0, be0, Wc1, bc1,
           g1, be1, Wc2, bc2, g2, be2, W_jk, b_jk, Wh1, bh1, Wh2, bh2):
    f32 = jnp.float32
    pad = EPAD - E
    src_p = jnp.concatenate(
        [edge_index[0], jnp.zeros((pad,), jnp.int32)]).reshape(NW, NCHUNK, CH)
    pad_dst = N + (jnp.arange(pad, dtype=jnp.int32) % (NPAD - N))
    dst_p = jnp.concatenate(
        [edge_index[1], pad_dst]).reshape(NW, NCHUNK, CH)
    zeros_h = jnp.zeros((NPAD, H), f32)
    ones_d = jnp.ones((CH, DW), f32)
    batch_row = batch.reshape(1, N)

    _sc_degree, _sc_aggregate = _sc_kernels()
    deg_parts = _sc_degree(dst_p, ones_d, zeros_h)
    h0, y0, dinv = _tc_call(
        _tc_front_body,
        (jax.ShapeDtypeStruct((N, H), f32),
         jax.ShapeDtypeStruct((N, H), f32),
         jax.ShapeDtypeStruct((N, 1), f32)),
        x, W_in, b_in.reshape(1, H), Wc0, deg_parts)

    convs = ((bc0, g0, be0, Wc1), (bc1, g1, be1, Wc2), (bc2, g2, be2, Wc2))
    h, y = h0, y0
    pools = []
    for li, (b, g, be, wnext) in enumerate(convs):
        parts = _sc_aggregate(y, src_p, dst_p, zeros_h)
        has_next = li < 2
        outs = _tc_call(
            functools.partial(_tc_layer_body, has_next),
            (jax.ShapeDtypeStruct((N, H), f32),
             jax.ShapeDtypeStruct((G, H), f32),
             jax.ShapeDtypeStruct((N, H), f32)),
            parts, y, h, dinv, b.reshape(1, H), g.reshape(1, H),
            be.reshape(1, H), batch_row, wnext)
        h, pool, y = outs
        pools.append(pool)

    out = _tc_call(
        _tc_head_body,
        jax.ShapeDtypeStruct((G, OUT), f32),
        pools[0], pools[1], pools[2], W_jk, b_jk.reshape(1, H),
        Wh1, bh1.reshape(1, H), Wh2, bh2.reshape(1, OUT))
    return out


# speedup vs baseline: 2.0078x; 1.3585x over previous
import functools

import jax
import jax.numpy as jnp
from jax import lax
from jax.experimental import pallas as pl
from jax.experimental.pallas import tpu as pltpu
from jax.experimental.pallas import tpu_sc as plsc

N = 10000
E = 320000
D = 128
H = 128
G = 128
OUT = 10
EPS = 1e-5

NC = 2
NS = 16
NW = NC * NS
CH = 128
PT = 10112
NCHUNK = PT // CH
FAST_C = 0
NCF = 99
NCS = 58
NCMAX = NCF
EPAD = PT * NW
NPAD = 10112
ZR = NPAD // NS
DW = 128

@functools.cache
def _sc_kernels():
    mesh = plsc.VectorSubcoreMesh(core_axis_name="c", subcore_axis_name="s",
                                  num_cores=NC, num_subcores=NS)
    deg = functools.partial(
        pl.kernel,
        out_type=jax.ShapeDtypeStruct((NC, NPAD, DW), jnp.float32),
        mesh=mesh,
        scratch_types=[
            pltpu.VMEM((NCHUNK, CH), jnp.int32),
            pltpu.VMEM((CH, DW), jnp.float32),
            pltpu.VMEM_SHARED((NPAD, DW), jnp.float32),
        ],
    )(_sc_degree_body)
    agg = functools.partial(
        pl.kernel,
        out_type=jax.ShapeDtypeStruct((NC, NPAD, H), jnp.float32),
        mesh=mesh,
        scratch_types=[
            pltpu.VMEM((NCMAX, CH), jnp.int32),
            pltpu.VMEM((NCMAX, CH), jnp.int32),
            pltpu.VMEM((CH, H), jnp.float32),
            pltpu.VMEM_SHARED((NPAD, H), jnp.float32),
            pltpu.SemaphoreType.DMA,
        ],
    )(_sc_aggregate_body)
    return deg, agg


def _sc_degree_body(dst_hbm, ones_hbm, zeros_hbm, out_hbm, dst_v, ones_v, acc):
    c = lax.axis_index("c")
    s = lax.axis_index("s")
    wid = s * NC + c
    pltpu.sync_copy(dst_hbm.at[wid], dst_v)
    pltpu.sync_copy(ones_hbm, ones_v)
    pltpu.sync_copy(zeros_hbm.at[pl.ds(s * ZR, ZR)], acc.at[pl.ds(s * ZR, ZR)])
    plsc.subcore_barrier()

    def body(j, carry):
        pltpu.sync_copy(ones_v, acc.at[dst_v.at[j]], add=True)
        return carry

    lax.fori_loop(0, NCHUNK, body, 0)
    plsc.subcore_barrier()
    pltpu.sync_copy(acc.at[pl.ds(s * ZR, ZR)], out_hbm.at[c].at[pl.ds(s * ZR, ZR)])


def _sc_aggregate_body(y_hbm, src_hbm, dst_hbm, zeros_hbm, out_hbm,
                       src_v, dst_v, buf_a, acc, sem_a):
    c = lax.axis_index("c")
    s = lax.axis_index("s")
    wid = s * NC + c
    pltpu.sync_copy(src_hbm.at[wid], src_v)
    pltpu.sync_copy(dst_hbm.at[wid], dst_v)
    pltpu.sync_copy(zeros_hbm.at[pl.ds(s * ZR, ZR)], acc.at[pl.ds(s * ZR, ZR)])
    plsc.subcore_barrier()

    def body(k, carry):
        pltpu.async_copy(y_hbm.at[src_v.at[k]], buf_a, sem_a).wait()
        pltpu.sync_copy(buf_a, acc.at[dst_v.at[k]], add=True)
        return carry

    nchunks = jnp.where(c == FAST_C, NCF, NCS)
    lax.fori_loop(0, nchunks, body, 0)
    plsc.subcore_barrier()
    pltpu.sync_copy(acc.at[pl.ds(s * ZR, ZR)], out_hbm.at[c].at[pl.ds(s * ZR, ZR)])


def _tc_front_body(x_ref, win_ref, bin_ref, wc0_ref, degp_ref,
                   h0_ref, y0_ref, dinv_ref):
    degp = degp_ref[...]
    deg = degp[0, :N, 0] + degp[1, :N, 0] + 1.0
    dinv = lax.rsqrt(jnp.maximum(deg, 1.0))
    h0 = jnp.maximum(x_ref[...] @ win_ref[...] + bin_ref[...], 0.0)
    h0_ref[...] = h0
    y0_ref[...] = (h0 @ wc0_ref[...]) * dinv[:, None]
    dinv_ref[...] = dinv[:, None]


def _tc_layer_body(has_next, parts_ref, y_ref, h_ref, dinv_ref, b_ref,
                   g_ref, be_ref, batch_ref, wnext_ref,
                   hn_ref, pool_ref, ynext_ref):
    parts = parts_ref[...]
    dinv = dinv_ref[...]
    agg = (parts[0, :N] + parts[1, :N] + y_ref[...]) * dinv + b_ref[...]
    mu = jnp.mean(agg, axis=0, keepdims=True)
    var = jnp.mean((agg - mu) ** 2, axis=0, keepdims=True)
    hnew = (agg - mu) * lax.rsqrt(var + EPS) * g_ref[...] + be_ref[...]
    hnew = h_ref[...] + jnp.maximum(hnew, 0.0)
    hn_ref[...] = hnew
    seg = jnp.arange(G, dtype=jnp.int32)[:, None] == batch_ref[...]
    pool_ref[...] = jnp.dot(seg.astype(jnp.float32), hnew,
                            preferred_element_type=jnp.float32)
    if has_next:
        ynext_ref[...] = (hnew @ wnext_ref[...]) * dinv


def _tc_head_body(p0_ref, p1_ref, p2_ref, wjk_ref, bjk_ref, wh1_ref,
                  bh1_ref, wh2_ref, bh2_ref, out_ref):
    z = jnp.concatenate([p0_ref[...], p1_ref[...], p2_ref[...]], axis=1)
    z = jnp.maximum(z @ wjk_ref[...] + bjk_ref[...], 0.0)
    z = jnp.maximum(z @ wh1_ref[...] + bh1_ref[...], 0.0)
    out_ref[...] = z @ wh2_ref[...] + bh2_ref[...]


def _tc_call(body, out_shapes, *args):
    return pl.pallas_call(
        body,
        out_shape=out_shapes,
    )(*args)


def kernel(x, edge_index, batch, W_in, b_in, Wc0, bc0, g0, be0, Wc1, bc1,
           g1, be1, Wc2, bc2, g2, be2, W_jk, b_jk, Wh1, bh1, Wh2, bh2):
    f32 = jnp.float32
    pad = EPAD - E
    pad_dst = N + (jnp.arange(pad, dtype=jnp.int32) % (NPAD - N))
    dst_p = jnp.concatenate(
        [edge_index[1], pad_dst]).reshape(NW, NCHUNK, CH)

    EF = NCF * CH * NS
    pad2 = EF + NCS * CH * NS - E
    pad_dst2 = N + (jnp.arange(pad2, dtype=jnp.int32) % (NPAD - N))
    src_f = edge_index[0][:EF].reshape(NS, NCF, CH)
    dst_f = edge_index[1][:EF].reshape(NS, NCF, CH)
    src_s = jnp.concatenate(
        [edge_index[0][EF:], jnp.zeros((pad2,), jnp.int32)]).reshape(NS, NCS, CH)
    dst_s = jnp.concatenate(
        [edge_index[1][EF:], pad_dst2]).reshape(NS, NCS, CH)

    def _interleave(f, s_):
        s_ = jnp.pad(s_, ((0, 0), (0, NCMAX - NCS), (0, 0)))
        pair = (f, s_) if FAST_C == 0 else (s_, f)
        return jnp.stack(pair, axis=1).reshape(NW, NCMAX, CH)

    src_a = _interleave(src_f, src_s)
    dst_a = _interleave(dst_f, dst_s)
    zeros_h = jnp.zeros((NPAD, H), f32)
    ones_d = jnp.ones((CH, DW), f32)
    batch_row = batch.reshape(1, N)

    _sc_degree, _sc_aggregate = _sc_kernels()
    deg_parts = _sc_degree(dst_p, ones_d, zeros_h)
    h0, y0, dinv = _tc_call(
        _tc_front_body,
        (jax.ShapeDtypeStruct((N, H), f32),
         jax.ShapeDtypeStruct((N, H), f32),
         jax.ShapeDtypeStruct((N, 1), f32)),
        x, W_in, b_in.reshape(1, H), Wc0, deg_parts)

    convs = ((bc0, g0, be0, Wc1), (bc1, g1, be1, Wc2), (bc2, g2, be2, Wc2))
    h, y = h0, y0
    pools = []
    for li, (b, g, be, wnext) in enumerate(convs):
        parts = _sc_aggregate(y, src_a, dst_a, zeros_h)
        has_next = li < 2
        outs = _tc_call(
            functools.partial(_tc_layer_body, has_next),
            (jax.ShapeDtypeStruct((N, H), f32),
             jax.ShapeDtypeStruct((G, H), f32),
             jax.ShapeDtypeStruct((N, H), f32)),
            parts, y, h, dinv, b.reshape(1, H), g.reshape(1, H),
            be.reshape(1, H), batch_row, wnext)
        h, pool, y = outs
        pools.append(pool)

    out = _tc_call(
        _tc_head_body,
        jax.ShapeDtypeStruct((G, OUT), f32),
        pools[0], pools[1], pools[2], W_jk, b_jk.reshape(1, H),
        Wh1, bh1.reshape(1, H), Wh2, bh2.reshape(1, OUT))
    return out
